# baseline (speedup 1.0000x reference)
import functools

import jax
import jax.numpy as jnp
from jax import lax
from jax.experimental import pallas as pl
from jax.experimental.pallas import tpu as pltpu
from jax.experimental.pallas import tpu_sc as plsc

F32 = jnp.float32
I32 = jnp.int32

T, D, E, R = 4096, 1024, 256, 64
B, K, G, M, N = 64, 2, 8, 32, 100000
S = K * G * M
KG = K * G
NW = 32
ROWS_PER_W = (B * S) // NW
CHUNK = 128
NCHUNK = ROWS_PER_W // CHUNK
HS_W = 16
HS_PER_W = (2 * B) // HS_W


def _sc_gather_body(nbr_hbm, cat_hbm, mem_hbm, hid_hbm, embs_out, rows_out,
                    idx_v, buf_v, idx2_v, buf2_v, sem):
    wid = lax.axis_index("s") * 2 + lax.axis_index("c")
    for t in range(NCHUNK):
        base = wid * ROWS_PER_W + t * CHUNK
        pltpu.sync_copy(nbr_hbm.at[pl.ds(base, CHUNK)], idx_v)
        pltpu.async_copy(mem_hbm.at[idx_v], buf_v, sem).wait()
        pltpu.sync_copy(buf_v, embs_out.at[pl.ds(base, CHUNK)])

    @pl.when(wid < HS_W)
    def _():
        hbase = wid * HS_PER_W
        pltpu.sync_copy(cat_hbm.at[pl.ds(hbase, HS_PER_W)], idx2_v)
        pltpu.async_copy(hid_hbm.at[idx2_v], buf2_v, sem).wait()
        pltpu.sync_copy(buf2_v, rows_out.at[pl.ds(hbase, HS_PER_W)])


@functools.cache
def _sc_gather_fn():
    mesh = plsc.VectorSubcoreMesh(core_axis_name="c", subcore_axis_name="s")
    return pl.kernel(
        _sc_gather_body,
        mesh=mesh,
        out_type=[
            jax.ShapeDtypeStruct((B * S, E), F32),
            jax.ShapeDtypeStruct((2 * B, D), F32),
        ],
        scratch_types=[
            pltpu.VMEM((CHUNK,), I32),
            pltpu.VMEM((CHUNK, E), F32),
            pltpu.VMEM((HS_PER_W,), I32),
            pltpu.VMEM((HS_PER_W, D), F32),
            pltpu.SemaphoreType.DMA,
        ],
    )


def _sc_gather(nbr_flat, cat_idx, memory_nodes, hidden_states):
    return _sc_gather_fn()(nbr_flat, cat_idx, memory_nodes, hidden_states)


def _k1_body(rows_ref, wq_ref, bq_ref, wcls_ref, bcls_ref, q_out, relp_out):
    rows = rows_ref[...]
    g = rows[0:B]
    r = rows[B:2 * B]
    q_out[...] = jnp.tanh(
        jnp.dot(g, wq_ref[...], preferred_element_type=F32) + bq_ref[...])
    logits = jnp.dot(r, wcls_ref[...], preferred_element_type=F32) + bcls_ref[...]
    mx = jnp.max(logits, axis=1, keepdims=True)
    e = jnp.exp(logits - mx)
    relp_out[...] = e / jnp.sum(e, axis=1, keepdims=True)


def _k1(hs_rows, W_q, b_q2, W_cls, b_cls2):
    return pl.pallas_call(
        _k1_body,
        out_shape=[
            jax.ShapeDtypeStruct((B, D), F32),
            jax.ShapeDtypeStruct((B, R), F32),
        ],
    )(hs_rows, W_q, b_q2, W_cls, b_cls2)


def _k2_body(embs_ref, q_ref, relp_ref, grp_ref, wn_ref, bn_ref, wg_ref,
             bg_ref, out_ref):
    embs = embs_ref[...]
    proj = jnp.tanh(
        jnp.dot(embs, wn_ref[...], preferred_element_type=F32) + bn_ref[...])
    q = q_ref[0]
    dot = lax.dot_general(q, proj, (((1,), (1,)), ((), ())),
                          preferred_element_type=F32)

    seg = lax.broadcasted_iota(I32, (1, S), 1) // M
    expT = (lax.broadcasted_iota(I32, (S, KG), 0) // M
            == lax.broadcasted_iota(I32, (S, KG), 1)).astype(F32)

    maxv = jnp.zeros((1, S), F32) - 1e30
    for i in range(KG):
        mi = jnp.max(jnp.where(seg == i, dot, -1e30))
        maxv = jnp.where(seg == i, mi, maxv)
    e1 = jnp.exp(dot - maxv)
    gsum = jnp.dot(e1, expT, preferred_element_type=F32)
    denom = lax.dot_general(gsum, expT, (((1,), (1,)), ((), ())),
                            preferred_element_type=F32)
    attn = e1 / denom

    grp = grp_ref[0]
    oneh = (lax.broadcasted_iota(I32, (R, KG), 0)
            == jnp.broadcast_to(grp, (R, KG))).astype(F32)
    p16 = jnp.dot(relp_ref[0], oneh, preferred_element_type=F32)
    p_slot = lax.dot_general(p16, expT, (((1,), (1,)), ((), ())),
                             preferred_element_type=F32)

    e2 = jnp.exp(attn * p_slot * 10.0)
    konehT = (lax.broadcasted_iota(I32, (S, K), 0) // (G * M)
              == lax.broadcasted_iota(I32, (S, K), 1)).astype(F32)
    ksum = jnp.dot(e2, konehT, preferred_element_type=F32)
    denom2 = lax.dot_general(ksum, konehT, (((1,), (1,)), ((), ())),
                             preferred_element_type=F32)
    coef = e2 / denom2 * (1.0 / (G * M * K))

    mask = (embs[:, 0:1] != 0.0).astype(F32)
    membs = embs * mask
    asc = jnp.dot(coef, membs, preferred_element_type=F32)
    out_ref[0] = jnp.tanh(
        jnp.dot(asc, wg_ref[...], preferred_element_type=F32) + bg_ref[...])


def _k2(embs, q3, relp3, grp3, W_nodes, b_n2, W_gnn, b_g2):
    return pl.pallas_call(
        _k2_body,
        grid=(B,),
        in_specs=[
            pl.BlockSpec((S, E), lambda b: (b, 0)),
            pl.BlockSpec((1, 1, D), lambda b: (b, 0, 0)),
            pl.BlockSpec((1, 1, R), lambda b: (b, 0, 0)),
            pl.BlockSpec((1, 1, KG), lambda b: (b, 0, 0)),
            pl.BlockSpec((E, D), lambda b: (0, 0)),
            pl.BlockSpec((1, D), lambda b: (0, 0)),
            pl.BlockSpec((E, D), lambda b: (0, 0)),
            pl.BlockSpec((1, D), lambda b: (0, 0)),
        ],
        out_specs=pl.BlockSpec((1, 1, D), lambda b: (b, 0, 0)),
        out_shape=jax.ShapeDtypeStruct((B, 1, D), F32),
        compiler_params=pltpu.CompilerParams(
            dimension_semantics=("arbitrary",)),
    )(embs, q3, relp3, grp3, W_nodes, b_n2, W_gnn, b_g2)


def _k3_body(hid_ref, gnn_sm, gcol_ref, grow_ref, upd_ref, out_ref,
             rows_v, sem):
    del hid_ref
    dup = (gcol_ref[...] == grow_ref[...]).astype(F32)
    upd = jnp.dot(dup, upd_ref[...], preferred_element_type=F32)
    for b in range(B):
        pltpu.make_async_copy(out_ref.at[pl.ds(gnn_sm[b], 1)],
                              rows_v.at[pl.ds(b, 1)], sem).start()
    for b in range(B):
        pltpu.make_async_copy(out_ref.at[pl.ds(gnn_sm[b], 1)],
                              rows_v.at[pl.ds(b, 1)], sem).wait()
    rows_v[...] = rows_v[...] + upd
    for b in range(B):
        pltpu.make_async_copy(rows_v.at[pl.ds(b, 1)],
                              out_ref.at[pl.ds(gnn_sm[b], 1)], sem).start()
    for b in range(B):
        pltpu.make_async_copy(rows_v.at[pl.ds(b, 1)],
                              out_ref.at[pl.ds(gnn_sm[b], 1)], sem).wait()


def _k3(hidden_states, gnn_i32, gnn_col, gnn_row, upd_rows):
    return pl.pallas_call(
        _k3_body,
        in_specs=[
            pl.BlockSpec(memory_space=pltpu.MemorySpace.HBM),
            pl.BlockSpec(memory_space=pltpu.MemorySpace.SMEM),
            pl.BlockSpec((B, 1), lambda: (0, 0)),
            pl.BlockSpec((1, B), lambda: (0, 0)),
            pl.BlockSpec((B, D), lambda: (0, 0)),
        ],
        out_specs=pl.BlockSpec(memory_space=pltpu.MemorySpace.HBM),
        out_shape=jax.ShapeDtypeStruct((T, D), F32),
        input_output_aliases={0: 0},
        scratch_shapes=[
            pltpu.VMEM((B, D), F32),
            pltpu.SemaphoreType.DMA,
        ],
    )(hidden_states, gnn_i32, gnn_col, gnn_row, upd_rows)


def kernel(hidden_states, memory_nodes, gnn_idx, rel_idx, neighbor_idx,
           group_rel_ids, W_cls, b_cls, W_q, b_q, W_nodes, b_nodes, W_gnn,
           b_gnn):
    nbr_flat = neighbor_idx.reshape(-1).astype(I32)
    cat_idx = jnp.concatenate([gnn_idx, rel_idx]).astype(I32)

    embs, hs_rows = _sc_gather(nbr_flat, cat_idx, memory_nodes, hidden_states)
    q, rel_prob = _k1(hs_rows, W_q, b_q.reshape(1, D), W_cls,
                      b_cls.reshape(1, R))
    out_rows = _k2(embs, q.reshape(B, 1, D), rel_prob.reshape(B, 1, R),
                   group_rel_ids.reshape(B, 1, KG).astype(I32),
                   W_nodes, b_nodes.reshape(1, D), W_gnn, b_gnn.reshape(1, D))
    gnn_i32 = gnn_idx.astype(I32)
    return _k3(hidden_states, gnn_i32, gnn_i32.reshape(B, 1),
               gnn_i32.reshape(1, B), out_rows.reshape(B, D))


# speedup vs baseline: 1.0731x; 1.0731x over previous
import functools

import jax
import jax.numpy as jnp
from jax import lax
from jax.experimental import pallas as pl
from jax.experimental.pallas import tpu as pltpu
from jax.experimental.pallas import tpu_sc as plsc

F32 = jnp.float32
BF16 = jnp.bfloat16
I32 = jnp.int32

T, D, E, R = 4096, 1024, 256, 64
B, K, G, M, N = 64, 2, 8, 32, 100000
S = K * G * M
KG = K * G
NW = 32
ROWS_PER_W = (B * S) // NW
CHUNK = 128
NCHUNK = ROWS_PER_W // CHUNK
NB = 2


def _sc_gather_body(nbr_hbm, mem_hbm, embs_out, idx_v, buf_v, sem):
    wid = lax.axis_index("s") * 2 + lax.axis_index("c")
    for t in range(NCHUNK):
        base = wid * ROWS_PER_W + t * CHUNK
        pltpu.sync_copy(nbr_hbm.at[pl.ds(base, CHUNK)], idx_v)
        pltpu.async_copy(mem_hbm.at[idx_v], buf_v, sem).wait()
        pltpu.sync_copy(buf_v, embs_out.at[pl.ds(base, CHUNK)])


@functools.cache
def _sc_gather_fn():
    mesh = plsc.VectorSubcoreMesh(core_axis_name="c", subcore_axis_name="s")
    return pl.kernel(
        _sc_gather_body,
        mesh=mesh,
        out_type=jax.ShapeDtypeStruct((B * S, E), F32),
        scratch_types=[
            pltpu.VMEM((CHUNK,), I32),
            pltpu.VMEM((CHUNK, E), F32),
            pltpu.SemaphoreType.DMA,
        ],
    )


def _sc_gather(nbr_flat, memory_nodes):
    return _sc_gather_fn()(nbr_flat, memory_nodes)


def _k1_body(cat_sm, hid_ref, wq_ref, bq_ref, wcls_ref, bcls_ref,
             q_out, relp_out, rows_v, sem):
    for i in range(2 * B):
        pltpu.make_async_copy(hid_ref.at[pl.ds(cat_sm[i], 1)],
                              rows_v.at[pl.ds(i, 1)], sem).start()
    for i in range(2 * B):
        pltpu.make_async_copy(hid_ref.at[pl.ds(cat_sm[i], 1)],
                              rows_v.at[pl.ds(i, 1)], sem).wait()
    rows = rows_v[...]
    g = rows[0:B]
    r = rows[B:2 * B]
    q_out[...] = jnp.tanh(
        jnp.dot(g, wq_ref[...], preferred_element_type=F32) + bq_ref[...])
    logits = jnp.dot(r, wcls_ref[...], preferred_element_type=F32) + bcls_ref[...]
    mx = jnp.max(logits, axis=1, keepdims=True)
    e = jnp.exp(logits - mx)
    relp_out[...] = e / jnp.sum(e, axis=1, keepdims=True)


def _k1(cat_idx, hidden_states, W_q, b_q2, W_cls, b_cls2):
    return pl.pallas_call(
        _k1_body,
        in_specs=[
            pl.BlockSpec(memory_space=pltpu.MemorySpace.SMEM),
            pl.BlockSpec(memory_space=pltpu.MemorySpace.HBM),
            pl.BlockSpec((D, D), lambda: (0, 0)),
            pl.BlockSpec((1, D), lambda: (0, 0)),
            pl.BlockSpec((D, R), lambda: (0, 0)),
            pl.BlockSpec((1, R), lambda: (0, 0)),
        ],
        out_shape=[
            jax.ShapeDtypeStruct((B, D), F32),
            jax.ShapeDtypeStruct((B, R), F32),
        ],
        scratch_shapes=[
            pltpu.VMEM((2 * B, D), F32),
            pltpu.SemaphoreType.DMA,
        ],
    )(cat_idx, hidden_states, W_q, b_q2, W_cls, b_cls2)


def _k2_body(embs_ref, q_ref, relp_ref, grp_ref, wn_ref, bn_ref, wg_ref,
             bg_ref, out_ref):
    expT = (lax.broadcasted_iota(I32, (S, KG), 0) // M
            == lax.broadcasted_iota(I32, (S, KG), 1)).astype(F32)
    konehT = (lax.broadcasted_iota(I32, (S, K), 0) // (G * M)
              == lax.broadcasted_iota(I32, (S, K), 1)).astype(F32)

    for j in range(NB):
        embs = embs_ref[pl.ds(j * S, S), :]
        proj = jnp.tanh(
            jnp.dot(embs.astype(BF16), wn_ref[...],
                    preferred_element_type=F32) + bn_ref[...])
        q = q_ref[j].astype(BF16)
        dot = lax.dot_general(q, proj.astype(BF16), (((1,), (1,)), ((), ())),
                              preferred_element_type=F32)

        e1 = jnp.exp(dot - jnp.max(dot))
        gsum = jnp.dot(e1, expT, preferred_element_type=F32)
        denom = lax.dot_general(gsum, expT, (((1,), (1,)), ((), ())),
                                preferred_element_type=F32)
        attn = e1 / denom

        grp = grp_ref[j]
        oneh = (lax.broadcasted_iota(I32, (R, KG), 0)
                == jnp.broadcast_to(grp, (R, KG))).astype(F32)
        p16 = jnp.dot(relp_ref[j], oneh, preferred_element_type=F32)
        p_slot = lax.dot_general(p16, expT, (((1,), (1,)), ((), ())),
                                 preferred_element_type=F32)

        e2 = jnp.exp(attn * p_slot * 10.0)
        ksum = jnp.dot(e2, konehT, preferred_element_type=F32)
        denom2 = lax.dot_general(ksum, konehT, (((1,), (1,)), ((), ())),
                                 preferred_element_type=F32)
        coef = (e2 / denom2 * (1.0 / (G * M * K))).astype(BF16)

        mask = (embs[:, 0:1] != 0.0).astype(BF16)
        membs = embs.astype(BF16) * mask
        asc = jnp.dot(coef, membs, preferred_element_type=F32)
        out_ref[j] = jnp.tanh(
            jnp.dot(asc.astype(BF16), wg_ref[...],
                    preferred_element_type=F32) + bg_ref[...])


def _k2(embs, q3, relp3, grp3, W_nodes, b_n2, W_gnn, b_g2):
    return pl.pallas_call(
        _k2_body,
        grid=(B // NB,),
        in_specs=[
            pl.BlockSpec((NB * S, E), lambda b: (b, 0)),
            pl.BlockSpec((NB, 1, D), lambda b: (b, 0, 0)),
            pl.BlockSpec((NB, 1, R), lambda b: (b, 0, 0)),
            pl.BlockSpec((NB, 1, KG), lambda b: (b, 0, 0)),
            pl.BlockSpec((E, D), lambda b: (0, 0)),
            pl.BlockSpec((1, D), lambda b: (0, 0)),
            pl.BlockSpec((E, D), lambda b: (0, 0)),
            pl.BlockSpec((1, D), lambda b: (0, 0)),
        ],
        out_specs=pl.BlockSpec((NB, 1, D), lambda b: (b, 0, 0)),
        out_shape=jax.ShapeDtypeStruct((B, 1, D), F32),
        compiler_params=pltpu.CompilerParams(
            dimension_semantics=("arbitrary",)),
    )(embs, q3, relp3, grp3, W_nodes, b_n2, W_gnn, b_g2)


def _k3_body(hid_ref, gnn_sm, gcol_ref, grow_ref, upd_ref, out_ref,
             rows_v, sem):
    del hid_ref
    dup = (gcol_ref[...] == grow_ref[...]).astype(F32)
    upd = jnp.dot(dup, upd_ref[...], preferred_element_type=F32)
    for b in range(B):
        pltpu.make_async_copy(out_ref.at[pl.ds(gnn_sm[b], 1)],
                              rows_v.at[pl.ds(b, 1)], sem).start()
    for b in range(B):
        pltpu.make_async_copy(out_ref.at[pl.ds(gnn_sm[b], 1)],
                              rows_v.at[pl.ds(b, 1)], sem).wait()
    rows_v[...] = rows_v[...] + upd
    for b in range(B):
        pltpu.make_async_copy(rows_v.at[pl.ds(b, 1)],
                              out_ref.at[pl.ds(gnn_sm[b], 1)], sem).start()
    for b in range(B):
        pltpu.make_async_copy(rows_v.at[pl.ds(b, 1)],
                              out_ref.at[pl.ds(gnn_sm[b], 1)], sem).wait()


def _k3(hidden_states, gnn_i32, gnn_col, gnn_row, upd_rows):
    return pl.pallas_call(
        _k3_body,
        in_specs=[
            pl.BlockSpec(memory_space=pltpu.MemorySpace.HBM),
            pl.BlockSpec(memory_space=pltpu.MemorySpace.SMEM),
            pl.BlockSpec((B, 1), lambda: (0, 0)),
            pl.BlockSpec((1, B), lambda: (0, 0)),
            pl.BlockSpec((B, D), lambda: (0, 0)),
        ],
        out_specs=pl.BlockSpec(memory_space=pltpu.MemorySpace.HBM),
        out_shape=jax.ShapeDtypeStruct((T, D), F32),
        input_output_aliases={0: 0},
        scratch_shapes=[
            pltpu.VMEM((B, D), F32),
            pltpu.SemaphoreType.DMA,
        ],
    )(hidden_states, gnn_i32, gnn_col, gnn_row, upd_rows)


def kernel(hidden_states, memory_nodes, gnn_idx, rel_idx, neighbor_idx,
           group_rel_ids, W_cls, b_cls, W_q, b_q, W_nodes, b_nodes, W_gnn,
           b_gnn):
    nbr_flat = neighbor_idx.reshape(-1).astype(I32)
    cat_idx = jnp.concatenate([gnn_idx, rel_idx]).astype(I32)

    embs = _sc_gather(nbr_flat, memory_nodes)
    q, rel_prob = _k1(cat_idx, hidden_states, W_q, b_q.reshape(1, D), W_cls,
                      b_cls.reshape(1, R))
    out_rows = _k2(embs, q.reshape(B, 1, D), rel_prob.reshape(B, 1, R),
                   group_rel_ids.reshape(B, 1, KG).astype(I32),
                   W_nodes.astype(BF16), b_nodes.reshape(1, D),
                   W_gnn.astype(BF16), b_gnn.reshape(1, D))
    gnn_i32 = gnn_idx.astype(I32)
    return _k3(hidden_states, gnn_i32, gnn_i32.reshape(B, 1),
               gnn_i32.reshape(1, B), out_rows.reshape(B, D))


# speedup vs baseline: 1.3296x; 1.2390x over previous
import functools

import jax
import jax.numpy as jnp
from jax import lax
from jax.experimental import pallas as pl
from jax.experimental.pallas import tpu as pltpu
from jax.experimental.pallas import tpu_sc as plsc

F32 = jnp.float32
BF16 = jnp.bfloat16
I32 = jnp.int32

T, D, E, R = 4096, 1024, 256, 64
B, K, G, M, N = 64, 2, 8, 32, 100000
S = K * G * M
KG = K * G
NW = 32
ROWS_PER_W = (B * S) // NW
CHUNK = 128
NCHUNK = ROWS_PER_W // CHUNK
NB = 4
HB = T // (B // NB)


def _sc_gather_body(nbr_hbm, mem_hbm, embs_out, idx_v, buf_v, sem):
    wid = lax.axis_index("s") * 2 + lax.axis_index("c")
    for t in range(NCHUNK):
        base = wid * ROWS_PER_W + t * CHUNK
        pltpu.sync_copy(nbr_hbm.at[pl.ds(base, CHUNK)], idx_v)
        pltpu.async_copy(mem_hbm.at[idx_v], buf_v, sem).wait()
        pltpu.sync_copy(buf_v, embs_out.at[pl.ds(base, CHUNK)])


@functools.cache
def _sc_gather_fn():
    mesh = plsc.VectorSubcoreMesh(core_axis_name="c", subcore_axis_name="s")
    return pl.kernel(
        _sc_gather_body,
        mesh=mesh,
        out_type=jax.ShapeDtypeStruct((B * S, E), F32),
        scratch_types=[
            pltpu.VMEM((CHUNK,), I32),
            pltpu.VMEM((CHUNK, E), F32),
            pltpu.SemaphoreType.DMA,
        ],
    )


def _sc_gather(nbr_flat, memory_nodes):
    return _sc_gather_fn()(nbr_flat, memory_nodes)


def _k1_body(cat_sm, hid_ref, wq_ref, bq_ref, wcls_ref, bcls_ref,
             q_out, relp_out, rows_v, sem):
    for i in range(2 * B):
        pltpu.make_async_copy(hid_ref.at[pl.ds(cat_sm[i], 1)],
                              rows_v.at[pl.ds(i, 1)], sem).start()
    for i in range(2 * B):
        pltpu.make_async_copy(hid_ref.at[pl.ds(cat_sm[i], 1)],
                              rows_v.at[pl.ds(i, 1)], sem).wait()
    rows = rows_v[...]
    g = rows[0:B]
    r = rows[B:2 * B]
    q_out[...] = jnp.tanh(
        jnp.dot(g, wq_ref[...], preferred_element_type=F32) + bq_ref[...])
    logits = jnp.dot(r, wcls_ref[...], preferred_element_type=F32) + bcls_ref[...]
    mx = jnp.max(logits, axis=1, keepdims=True)
    e = jnp.exp(logits - mx)
    relp_out[...] = e / jnp.sum(e, axis=1, keepdims=True)


def _k1(cat_idx, hidden_states, W_q, b_q2, W_cls, b_cls2):
    return pl.pallas_call(
        _k1_body,
        in_specs=[
            pl.BlockSpec(memory_space=pltpu.MemorySpace.SMEM),
            pl.BlockSpec(memory_space=pltpu.MemorySpace.HBM),
            pl.BlockSpec((D, D), lambda: (0, 0)),
            pl.BlockSpec((1, D), lambda: (0, 0)),
            pl.BlockSpec((D, R), lambda: (0, 0)),
            pl.BlockSpec((1, R), lambda: (0, 0)),
        ],
        out_shape=[
            jax.ShapeDtypeStruct((B, D), F32),
            jax.ShapeDtypeStruct((B, R), F32),
        ],
        scratch_shapes=[
            pltpu.VMEM((2 * B, D), F32),
            pltpu.SemaphoreType.DMA,
        ],
    )(cat_idx, hidden_states, W_q, b_q2, W_cls, b_cls2)


def _k2_body(embs_ref, q_ref, relp_ref, grp_ref, wn_ref, bn_ref, wg_ref,
             bg_ref, expT_ref, konehT_ref, hid_ref, row_out, hid_out):
    hid_out[...] = hid_ref[...]
    expT = expT_ref[...]
    konehT = konehT_ref[...]

    proj = []
    for j in range(NB):
        embs = embs_ref[pl.ds(j * S, S), :]
        proj.append(jnp.tanh(
            jnp.dot(embs.astype(BF16), wn_ref[...],
                    preferred_element_type=F32) + bn_ref[...]).astype(BF16))

    dots = []
    for j in range(NB):
        q = q_ref[j].astype(BF16)
        dots.append(lax.dot_general(q, proj[j], (((1,), (1,)), ((), ())),
                                    preferred_element_type=F32))

    coefs = []
    for j in range(NB):
        dot = dots[j]
        e1 = jnp.exp(dot - jnp.max(dot))
        gsum = jnp.dot(e1, expT, preferred_element_type=F32)
        denom = lax.dot_general(gsum, expT, (((1,), (1,)), ((), ())),
                                preferred_element_type=F32)
        attn = e1 / denom

        grp = grp_ref[j]
        oneh = (lax.broadcasted_iota(I32, (R, KG), 0)
                == jnp.broadcast_to(grp, (R, KG))).astype(F32)
        p16 = jnp.dot(relp_ref[j], oneh, preferred_element_type=F32)
        p_slot = lax.dot_general(p16, expT, (((1,), (1,)), ((), ())),
                                 preferred_element_type=F32)

        e2 = jnp.exp(attn * p_slot * 10.0)
        ksum = jnp.dot(e2, konehT, preferred_element_type=F32)
        denom2 = lax.dot_general(ksum, konehT, (((1,), (1,)), ((), ())),
                                 preferred_element_type=F32)
        coefs.append((e2 / denom2 * (1.0 / (G * M * K))).astype(BF16))

    for j in range(NB):
        embs = embs_ref[pl.ds(j * S, S), :]
        mask = (embs[:, 0:1] != 0.0).astype(BF16)
        membs = embs.astype(BF16) * mask
        asc = jnp.dot(coefs[j], membs, preferred_element_type=F32)
        row_out[j] = jnp.tanh(
            jnp.dot(asc.astype(BF16), wg_ref[...],
                    preferred_element_type=F32) + bg_ref[...])


def _k2(embs, q3, relp3, grp3, W_nodes, b_n2, W_gnn, b_g2, expT, konehT,
        hidden_states):
    return pl.pallas_call(
        _k2_body,
        grid=(B // NB,),
        in_specs=[
            pl.BlockSpec((NB * S, E), lambda b: (b, 0)),
            pl.BlockSpec((NB, 1, D), lambda b: (b, 0, 0)),
            pl.BlockSpec((NB, 1, R), lambda b: (b, 0, 0)),
            pl.BlockSpec((NB, 1, KG), lambda b: (b, 0, 0)),
            pl.BlockSpec((E, D), lambda b: (0, 0)),
            pl.BlockSpec((1, D), lambda b: (0, 0)),
            pl.BlockSpec((E, D), lambda b: (0, 0)),
            pl.BlockSpec((1, D), lambda b: (0, 0)),
            pl.BlockSpec((S, KG), lambda b: (0, 0)),
            pl.BlockSpec((S, K), lambda b: (0, 0)),
            pl.BlockSpec((HB, D), lambda b: (b, 0)),
        ],
        out_specs=[
            pl.BlockSpec((NB, 1, D), lambda b: (b, 0, 0)),
            pl.BlockSpec((HB, D), lambda b: (b, 0)),
        ],
        out_shape=[
            jax.ShapeDtypeStruct((B, 1, D), F32),
            jax.ShapeDtypeStruct((T, D), F32),
        ],
        compiler_params=pltpu.CompilerParams(
            dimension_semantics=("arbitrary",)),
    )(embs, q3, relp3, grp3, W_nodes, b_n2, W_gnn, b_g2, expT, konehT,
      hidden_states)


def _k3_body(hid_ref, gnn_sm, gcol_ref, grow_ref, upd_ref, out_ref,
             rows_v, sem):
    del hid_ref
    dup = (gcol_ref[...] == grow_ref[...]).astype(F32)
    upd = jnp.dot(dup, upd_ref[...], preferred_element_type=F32)
    for b in range(B):
        pltpu.make_async_copy(out_ref.at[pl.ds(gnn_sm[b], 1)],
                              rows_v.at[pl.ds(b, 1)], sem).start()
    for b in range(B):
        pltpu.make_async_copy(out_ref.at[pl.ds(gnn_sm[b], 1)],
                              rows_v.at[pl.ds(b, 1)], sem).wait()
    rows_v[...] = rows_v[...] + upd
    for b in range(B):
        pltpu.make_async_copy(rows_v.at[pl.ds(b, 1)],
                              out_ref.at[pl.ds(gnn_sm[b], 1)], sem).start()
    for b in range(B):
        pltpu.make_async_copy(rows_v.at[pl.ds(b, 1)],
                              out_ref.at[pl.ds(gnn_sm[b], 1)], sem).wait()


def _k3(new_hidden, gnn_i32, gnn_col, gnn_row, upd_rows):
    return pl.pallas_call(
        _k3_body,
        in_specs=[
            pl.BlockSpec(memory_space=pltpu.MemorySpace.HBM),
            pl.BlockSpec(memory_space=pltpu.MemorySpace.SMEM),
            pl.BlockSpec((B, 1), lambda: (0, 0)),
            pl.BlockSpec((1, B), lambda: (0, 0)),
            pl.BlockSpec((B, D), lambda: (0, 0)),
        ],
        out_specs=pl.BlockSpec(memory_space=pltpu.MemorySpace.HBM),
        out_shape=jax.ShapeDtypeStruct((T, D), F32),
        input_output_aliases={0: 0},
        scratch_shapes=[
            pltpu.VMEM((B, D), F32),
            pltpu.SemaphoreType.DMA,
        ],
    )(new_hidden, gnn_i32, gnn_col, gnn_row, upd_rows)


def kernel(hidden_states, memory_nodes, gnn_idx, rel_idx, neighbor_idx,
           group_rel_ids, W_cls, b_cls, W_q, b_q, W_nodes, b_nodes, W_gnn,
           b_gnn):
    nbr_flat = neighbor_idx.reshape(-1).astype(I32)
    cat_idx = jnp.concatenate([gnn_idx, rel_idx]).astype(I32)
    slots = jnp.arange(S, dtype=I32)
    expT = (slots[:, None] // M == jnp.arange(KG, dtype=I32)[None, :]
            ).astype(F32)
    konehT = (slots[:, None] // (G * M) == jnp.arange(K, dtype=I32)[None, :]
              ).astype(F32)

    embs = _sc_gather(nbr_flat, memory_nodes)
    q, rel_prob = _k1(cat_idx, hidden_states, W_q, b_q.reshape(1, D), W_cls,
                      b_cls.reshape(1, R))
    out_rows, new_hidden = _k2(
        embs, q.reshape(B, 1, D), rel_prob.reshape(B, 1, R),
        group_rel_ids.reshape(B, 1, KG).astype(I32),
        W_nodes.astype(BF16), b_nodes.reshape(1, D),
        W_gnn.astype(BF16), b_gnn.reshape(1, D), expT, konehT, hidden_states)
    gnn_i32 = gnn_idx.astype(I32)
    return _k3(new_hidden, gnn_i32, gnn_i32.reshape(B, 1),
               gnn_i32.reshape(1, B), out_rows.reshape(B, D))


# speedup vs baseline: 1.3649x; 1.0266x over previous
import functools

import jax
import jax.numpy as jnp
from jax import lax
from jax.experimental import pallas as pl
from jax.experimental.pallas import tpu as pltpu
from jax.experimental.pallas import tpu_sc as plsc

F32 = jnp.float32
BF16 = jnp.bfloat16
I32 = jnp.int32

T, D, E, R = 4096, 1024, 256, 64
B, K, G, M, N = 64, 2, 8, 32, 100000
S = K * G * M
KG = K * G
NW = 32
CHUNK = 128
NB = 8
NSPLIT = 2
BH = B // NSPLIT
NSTEP = BH // NB
HB = T // (NSTEP * NSPLIT)


_SC_ROWS = (BH * S) // NW
_SC_NCHUNK = _SC_ROWS // CHUNK


def _sc_gather_body(nbr_hbm, mem_hbm, embs_out, idx_v, buf_v, sem):
    wid = lax.axis_index("s") * 2 + lax.axis_index("c")
    for t in range(_SC_NCHUNK):
        base = wid * _SC_ROWS + t * CHUNK
        pltpu.sync_copy(nbr_hbm.at[pl.ds(base, CHUNK)], idx_v)
        pltpu.async_copy(mem_hbm.at[idx_v], buf_v, sem).wait()
        pltpu.sync_copy(buf_v, embs_out.at[pl.ds(base, CHUNK)])


@functools.cache
def _sc_gather_fn():
    mesh = plsc.VectorSubcoreMesh(core_axis_name="c", subcore_axis_name="s")
    return pl.kernel(
        _sc_gather_body,
        mesh=mesh,
        out_type=jax.ShapeDtypeStruct((BH * S, E), F32),
        scratch_types=[
            pltpu.VMEM((CHUNK,), I32),
            pltpu.VMEM((CHUNK, E), F32),
            pltpu.SemaphoreType.DMA,
        ],
    )


def _sc_gather(nbr_flat, memory_nodes):
    return _sc_gather_fn()(nbr_flat, memory_nodes)


def _k1_body(cat_sm, hid_ref, wq_ref, bq_ref, wcls_ref, bcls_ref,
             q_out, relp_out, rows_v, sem):
    for i in range(2 * B):
        pltpu.make_async_copy(hid_ref.at[pl.ds(cat_sm[i], 1)],
                              rows_v.at[pl.ds(i, 1)], sem).start()
    for i in range(2 * B):
        pltpu.make_async_copy(hid_ref.at[pl.ds(cat_sm[i], 1)],
                              rows_v.at[pl.ds(i, 1)], sem).wait()
    rows = rows_v[...]
    g = rows[0:B]
    r = rows[B:2 * B]
    q_out[...] = jnp.tanh(
        jnp.dot(g, wq_ref[...], preferred_element_type=F32) + bq_ref[...])
    logits = jnp.dot(r, wcls_ref[...], preferred_element_type=F32) + bcls_ref[...]
    mx = jnp.max(logits, axis=1, keepdims=True)
    e = jnp.exp(logits - mx)
    relp_out[...] = e / jnp.sum(e, axis=1, keepdims=True)


def _k1(cat_idx, hidden_states, W_q, b_q2, W_cls, b_cls2):
    return pl.pallas_call(
        _k1_body,
        in_specs=[
            pl.BlockSpec(memory_space=pltpu.MemorySpace.SMEM),
            pl.BlockSpec(memory_space=pltpu.MemorySpace.HBM),
            pl.BlockSpec((D, D), lambda: (0, 0)),
            pl.BlockSpec((1, D), lambda: (0, 0)),
            pl.BlockSpec((D, R), lambda: (0, 0)),
            pl.BlockSpec((1, R), lambda: (0, 0)),
        ],
        out_shape=[
            jax.ShapeDtypeStruct((B, D), F32),
            jax.ShapeDtypeStruct((B, R), F32),
        ],
        scratch_shapes=[
            pltpu.VMEM((2 * B, D), F32),
            pltpu.SemaphoreType.DMA,
        ],
    )(cat_idx, hidden_states, W_q, b_q2, W_cls, b_cls2)


def _k2_body(embs_ref, q_ref, relp_ref, grp_ref, wn_ref, bn_ref, wg_ref,
             bg_ref, expT_ref, konehT_ref, hid_ref, row_out, hid_out):
    hid_out[...] = hid_ref[...]
    expT = expT_ref[...]
    konehT = konehT_ref[...]

    proj = []
    for j in range(NB):
        embs = embs_ref[pl.ds(j * S, S), :]
        proj.append(jnp.tanh(
            jnp.dot(embs.astype(BF16), wn_ref[...],
                    preferred_element_type=F32) + bn_ref[...]).astype(BF16))

    dots = []
    for j in range(NB):
        q = q_ref[j].astype(BF16)
        dots.append(lax.dot_general(q, proj[j], (((1,), (1,)), ((), ())),
                                    preferred_element_type=F32))

    coefs = []
    for j in range(NB):
        dot = dots[j]
        e1 = jnp.exp(dot - jnp.max(dot))
        gsum = jnp.dot(e1, expT, preferred_element_type=F32)
        denom = lax.dot_general(gsum, expT, (((1,), (1,)), ((), ())),
                                preferred_element_type=F32)
        attn = e1 / denom

        grp = grp_ref[j]
        oneh = (lax.broadcasted_iota(I32, (R, KG), 0)
                == jnp.broadcast_to(grp, (R, KG))).astype(F32)
        p16 = jnp.dot(relp_ref[j], oneh, preferred_element_type=F32)
        p_slot = lax.dot_general(p16, expT, (((1,), (1,)), ((), ())),
                                 preferred_element_type=F32)

        e2 = jnp.exp(attn * p_slot * 10.0)
        ksum = jnp.dot(e2, konehT, preferred_element_type=F32)
        denom2 = lax.dot_general(ksum, konehT, (((1,), (1,)), ((), ())),
                                 preferred_element_type=F32)
        coefs.append((e2 / denom2 * (1.0 / (G * M * K))).astype(BF16))

    for j in range(NB):
        embs = embs_ref[pl.ds(j * S, S), :]
        mask = (embs[:, 0:1] != 0.0).astype(BF16)
        membs = embs.astype(BF16) * mask
        asc = jnp.dot(coefs[j], membs, preferred_element_type=F32)
        row_out[j] = jnp.tanh(
            jnp.dot(asc.astype(BF16), wg_ref[...],
                    preferred_element_type=F32) + bg_ref[...])


def _k2_chunk(c, embs, q3, relp3, grp3, W_nodes, b_n2, W_gnn, b_g2, expT,
              konehT, hidden_states, prev_newhid=None):
    off = c * NSTEP
    in_specs = [
        pl.BlockSpec((NB * S, E), lambda b: (b, 0)),
        pl.BlockSpec((NB, 1, D), lambda b: (b, 0, 0)),
        pl.BlockSpec((NB, 1, R), lambda b: (b, 0, 0)),
        pl.BlockSpec((NB, 1, KG), lambda b: (b, 0, 0)),
        pl.BlockSpec((E, D), lambda b: (0, 0)),
        pl.BlockSpec((1, D), lambda b: (0, 0)),
        pl.BlockSpec((E, D), lambda b: (0, 0)),
        pl.BlockSpec((1, D), lambda b: (0, 0)),
        pl.BlockSpec((S, KG), lambda b: (0, 0)),
        pl.BlockSpec((S, K), lambda b: (0, 0)),
        pl.BlockSpec((HB, D), lambda b, off=off: (b + off, 0)),
    ]
    args = [embs, q3, relp3, grp3, W_nodes, b_n2, W_gnn, b_g2, expT, konehT,
            hidden_states]
    aliases = {}
    body = _k2_body
    if prev_newhid is not None:
        in_specs.append(pl.BlockSpec(memory_space=pltpu.MemorySpace.HBM))
        args.append(prev_newhid)
        aliases = {11: 1}

        def body(*refs):
            _k2_body(*refs[:11], *refs[12:])

    return pl.pallas_call(
        body,
        grid=(NSTEP,),
        in_specs=in_specs,
        out_specs=[
            pl.BlockSpec((NB, 1, D), lambda b: (b, 0, 0)),
            pl.BlockSpec((HB, D), lambda b, off=off: (b + off, 0)),
        ],
        out_shape=[
            jax.ShapeDtypeStruct((BH, 1, D), F32),
            jax.ShapeDtypeStruct((T, D), F32),
        ],
        input_output_aliases=aliases,
        compiler_params=pltpu.CompilerParams(
            dimension_semantics=("arbitrary",)),
    )(*args)


def _k3_body(hid_ref, gnn_sm, gcol_ref, grow_ref, upd_ref, out_ref,
             rows_v, sem):
    del hid_ref
    dup = (gcol_ref[...] == grow_ref[...]).astype(F32)
    upd = jnp.dot(dup, upd_ref[...], preferred_element_type=F32)
    for b in range(B):
        pltpu.make_async_copy(out_ref.at[pl.ds(gnn_sm[b], 1)],
                              rows_v.at[pl.ds(b, 1)], sem).start()
    for b in range(B):
        pltpu.make_async_copy(out_ref.at[pl.ds(gnn_sm[b], 1)],
                              rows_v.at[pl.ds(b, 1)], sem).wait()
    rows_v[...] = rows_v[...] + upd
    for b in range(B):
        pltpu.make_async_copy(rows_v.at[pl.ds(b, 1)],
                              out_ref.at[pl.ds(gnn_sm[b], 1)], sem).start()
    for b in range(B):
        pltpu.make_async_copy(rows_v.at[pl.ds(b, 1)],
                              out_ref.at[pl.ds(gnn_sm[b], 1)], sem).wait()


def _k3(new_hidden, gnn_i32, gnn_col, gnn_row, upd_rows):
    return pl.pallas_call(
        _k3_body,
        in_specs=[
            pl.BlockSpec(memory_space=pltpu.MemorySpace.HBM),
            pl.BlockSpec(memory_space=pltpu.MemorySpace.SMEM),
            pl.BlockSpec((B, 1), lambda: (0, 0)),
            pl.BlockSpec((1, B), lambda: (0, 0)),
            pl.BlockSpec((B, D), lambda: (0, 0)),
        ],
        out_specs=pl.BlockSpec(memory_space=pltpu.MemorySpace.HBM),
        out_shape=jax.ShapeDtypeStruct((T, D), F32),
        input_output_aliases={0: 0},
        scratch_shapes=[
            pltpu.VMEM((B, D), F32),
            pltpu.SemaphoreType.DMA,
        ],
    )(new_hidden, gnn_i32, gnn_col, gnn_row, upd_rows)


def kernel(hidden_states, memory_nodes, gnn_idx, rel_idx, neighbor_idx,
           group_rel_ids, W_cls, b_cls, W_q, b_q, W_nodes, b_nodes, W_gnn,
           b_gnn):
    nbr_flat = neighbor_idx.reshape(-1).astype(I32)
    cat_idx = jnp.concatenate([gnn_idx, rel_idx]).astype(I32)
    slots = jnp.arange(S, dtype=I32)
    expT = (slots[:, None] // M == jnp.arange(KG, dtype=I32)[None, :]
            ).astype(F32)
    konehT = (slots[:, None] // (G * M) == jnp.arange(K, dtype=I32)[None, :]
              ).astype(F32)

    q, rel_prob = _k1(cat_idx, hidden_states, W_q, b_q.reshape(1, D), W_cls,
                      b_cls.reshape(1, R))
    q3 = q.reshape(B, 1, D)
    relp3 = rel_prob.reshape(B, 1, R)
    grp3 = group_rel_ids.reshape(B, 1, KG).astype(I32)
    wn = W_nodes.astype(BF16)
    wg = W_gnn.astype(BF16)
    bn = b_nodes.reshape(1, D)
    bg = b_gnn.reshape(1, D)

    rows_parts = []
    new_hidden = None
    for c in range(NSPLIT):
        embs_c = _sc_gather(nbr_flat[c * BH * S:(c + 1) * BH * S],
                            memory_nodes)
        rows_c, new_hidden = _k2_chunk(
            c, embs_c, q3[c * BH:(c + 1) * BH], relp3[c * BH:(c + 1) * BH],
            grp3[c * BH:(c + 1) * BH], wn, bn, wg, bg, expT, konehT,
            hidden_states, new_hidden)
        rows_parts.append(rows_c.reshape(BH, D))
    out_rows = jnp.concatenate(rows_parts, axis=0)

    gnn_i32 = gnn_idx.astype(I32)
    return _k3(new_hidden, gnn_i32, gnn_i32.reshape(B, 1),
               gnn_i32.reshape(1, B), out_rows)
